# Initial kernel scaffold; baseline (speedup 1.0000x reference)
import jax
import jax.numpy as jnp
from jax.experimental import pallas as pl


def kernel(h, edge_index, user_context_features, W_self, W_neigh, b_sage, W_msg, b_msg, ln_gamma, ln_beta, W_gate, b_gate):
    raise NotImplementedError("write your pallas kernel here")


# speedup vs baseline: 4.5831x; 4.5831x over previous
import functools

import jax
import jax.numpy as jnp
from jax import lax
from jax.experimental import pallas as pl
from jax.experimental.pallas import tpu as pltpu
from jax.experimental.pallas import tpu_sc as plsc

NS = 16
CHUNK = 128
HF = 64


def _msg_body(h_ref, uc_ref, w1_ref, w2_ref, b_ref, g_ref, bt_ref,
              o0_ref, o1_ref):
    x = (jnp.dot(h_ref[...], w1_ref[...], preferred_element_type=jnp.float32)
         + jnp.dot(uc_ref[...], w2_ref[...], preferred_element_type=jnp.float32)
         + b_ref[...])
    mu = jnp.mean(x, axis=1, keepdims=True)
    var = jnp.mean((x - mu) ** 2, axis=1, keepdims=True)
    y = (x - mu) / jnp.sqrt(var + 1e-5) * g_ref[...] + bt_ref[...]
    y = jnp.maximum(y, 0.0)
    o0_ref[...] = y[:, :HF]
    o1_ref[...] = y[:, HF:]


def _fin_body(h_ref, a0_ref, a1_ref, a2_ref, a3_ref, dg_ref,
              ws_ref, wn_ref, bs_ref, wg1_ref, wg2_ref, bg_ref, o_ref):
    deg = dg_ref[...][:, 0:1]
    inv = 1.0 / jnp.where(deg > 0.0, deg, 1.0)
    h = h_ref[...]
    hn = jnp.concatenate([a0_ref[...], a1_ref[...]], axis=1) * inv
    conv = jnp.concatenate([a2_ref[...], a3_ref[...]], axis=1) * inv
    std = (jnp.dot(h, ws_ref[...], preferred_element_type=jnp.float32)
           + jnp.dot(hn, wn_ref[...], preferred_element_type=jnp.float32)
           + bs_ref[...])
    gate = jax.nn.sigmoid(
        jnp.dot(std, wg1_ref[...], preferred_element_type=jnp.float32)
        + jnp.dot(conv, wg2_ref[...], preferred_element_type=jnp.float32)
        + bg_ref[...])
    o_ref[...] = gate * std + (1.0 - gate) * conv


def _sc_scatter(src, dst, t0, t1, t2, t3, zrows, zdeg, ones8,
                *, n_pad, k_chunks):
    rpt = n_pad // NS
    edges_per_tile = k_chunks * CHUNK
    mesh = plsc.VectorSubcoreMesh(core_axis_name="c", subcore_axis_name="s")
    out_sds = jax.ShapeDtypeStruct((n_pad, HF), jnp.float32)

    @functools.partial(
        pl.kernel,
        mesh=mesh,
        out_type=[out_sds, out_sds, out_sds, out_sds,
                  jax.ShapeDtypeStruct((n_pad, 8), jnp.float32)],
        scratch_types=[
            pltpu.VMEM((CHUNK,), jnp.int32),
            pltpu.VMEM((CHUNK,), jnp.int32),
            pltpu.VMEM((CHUNK, HF), jnp.float32),
            pltpu.VMEM((CHUNK, 8), jnp.float32),
            pltpu.VMEM((rpt, HF), jnp.float32),
            pltpu.VMEM((rpt, 8), jnp.float32),
            pltpu.VMEM_SHARED((n_pad, HF), jnp.float32),
            pltpu.VMEM_SHARED((n_pad, 8), jnp.float32),
            pltpu.SemaphoreType.DMA,
        ],
        compiler_params=pltpu.CompilerParams(use_tc_tiling_on_sc=False),
    )
    def k(src_hbm, dst_hbm, t0_hbm, t1_hbm, t2_hbm, t3_hbm,
          zrows_hbm, zdeg_hbm, ones_hbm,
          out0, out1, out2, out3, out_deg,
          sidx_v, didx_v, rows_v, ones_v, zbuf_v, dbuf_v, acc_sh, deg_sh, sem):
        cid = lax.axis_index("c")
        sid = lax.axis_index("s")
        row0 = sid * rpt
        ebase = sid * edges_per_tile
        pltpu.sync_copy(ones_hbm, ones_v)
        pltpu.sync_copy(zdeg_hbm, dbuf_v)
        pltpu.sync_copy(dbuf_v, deg_sh.at[pl.ds(row0, rpt)])

        for p, (ta, tb, out) in enumerate(((t0_hbm, t2_hbm, (out0, out2)),
                                           (t1_hbm, t3_hbm, (out1, out3)))):
            pltpu.sync_copy(zrows_hbm, zbuf_v)
            pltpu.sync_copy(zbuf_v, acc_sh.at[pl.ds(row0, rpt)])
            plsc.subcore_barrier()

            def body(kk, carry, ta=ta, tb=tb, p=p):
                base = ebase + kk * CHUNK
                pltpu.sync_copy(src_hbm.at[pl.ds(base, CHUNK)], sidx_v)
                pltpu.sync_copy(dst_hbm.at[pl.ds(base, CHUNK)], didx_v)

                @pl.when(cid == 0)
                def _():
                    pltpu.async_copy(ta.at[sidx_v], rows_v, sem).wait()

                @pl.when(cid == 1)
                def _():
                    pltpu.async_copy(tb.at[sidx_v], rows_v, sem).wait()

                pltpu.sync_copy(rows_v, acc_sh.at[didx_v], add=True)
                if p == 0:
                    @pl.when(cid == 0)
                    def _():
                        pltpu.sync_copy(ones_v, deg_sh.at[didx_v], add=True)
                return carry

            lax.fori_loop(0, k_chunks, body, 0)
            plsc.subcore_barrier()

            pltpu.sync_copy(acc_sh.at[pl.ds(row0, rpt)], zbuf_v)

            @pl.when(cid == 0)
            def _(out=out):
                pltpu.sync_copy(zbuf_v, out[0].at[pl.ds(row0, rpt)])

            @pl.when(cid == 1)
            def _(out=out):
                pltpu.sync_copy(zbuf_v, out[1].at[pl.ds(row0, rpt)])

            if p == 0:
                @pl.when(cid == 0)
                def _():
                    pltpu.sync_copy(deg_sh.at[pl.ds(row0, rpt)], dbuf_v)
                    pltpu.sync_copy(dbuf_v, out_deg.at[pl.ds(row0, rpt)])
                plsc.subcore_barrier()

    return k(src, dst, t0, t1, t2, t3, zrows, zdeg, ones8)


def kernel(h, edge_index, user_context_features, W_self, W_neigh, b_sage,
           W_msg, b_msg, ln_gamma, ln_beta, W_gate, b_gate):
    n, f = h.shape
    e = edge_index.shape[1]
    assert f == 128

    k_chunks = -(-e // (NS * CHUNK))
    e_pad = NS * CHUNK * k_chunks
    src = edge_index[0].astype(jnp.int32)
    dst = edge_index[1].astype(jnp.int32)
    if e_pad > e:
        pad = e_pad - e
        src = jnp.concatenate([src, jnp.zeros((pad,), jnp.int32)])
        dst = jnp.concatenate([dst, jnp.full((pad,), n, jnp.int32)])
    n_pad = -(-(n + 1) // 128) * 128
    rpt = n_pad // NS

    blk = 1000
    grid = n // blk
    full = lambda r, c: pl.BlockSpec((r, c), lambda i: (0, 0))
    rowsb = lambda c: pl.BlockSpec((blk, c), lambda i: (i, 0))
    m0, m1 = pl.pallas_call(
        _msg_body,
        grid=(grid,),
        in_specs=[rowsb(128), rowsb(128), full(128, 128), full(128, 128),
                  full(1, 128), full(1, 128), full(1, 128)],
        out_specs=[rowsb(HF), rowsb(HF)],
        out_shape=[jax.ShapeDtypeStruct((n, HF), jnp.float32)] * 2,
    )(h, user_context_features, W_msg[:128], W_msg[128:],
      b_msg.reshape(1, 128), ln_gamma.reshape(1, 128), ln_beta.reshape(1, 128))

    zrows = jnp.zeros((rpt, HF), jnp.float32)
    zdeg = jnp.zeros((rpt, 8), jnp.float32)
    ones8 = jnp.ones((CHUNK, 8), jnp.float32)
    a0, a1, a2, a3, deg2 = _sc_scatter(
        src, dst, h[:, :HF], h[:, HF:], m0, m1, zrows, zdeg, ones8,
        n_pad=n_pad, k_chunks=k_chunks)

    out = pl.pallas_call(
        _fin_body,
        grid=(grid,),
        in_specs=[rowsb(128), rowsb(HF), rowsb(HF), rowsb(HF), rowsb(HF),
                  rowsb(8),
                  full(128, 128), full(128, 128), full(1, 128),
                  full(128, 128), full(128, 128), full(1, 128)],
        out_specs=rowsb(128),
        out_shape=jax.ShapeDtypeStruct((n, 128), jnp.float32),
    )(h, a0, a1, a2, a3, deg2,
      W_self, W_neigh, b_sage.reshape(1, 128),
      W_gate[:128], W_gate[128:], b_gate.reshape(1, 128))
    return out


# speedup vs baseline: 5.0720x; 1.1067x over previous
import functools

import jax
import jax.numpy as jnp
from jax import lax
from jax.experimental import pallas as pl
from jax.experimental.pallas import tpu as pltpu
from jax.experimental.pallas import tpu_sc as plsc

NS = 16
CHUNK = 128
HF = 64


def _msg_body(h_ref, uc_ref, w1_ref, w2_ref, b_ref, g_ref, bt_ref,
              o0_ref, o1_ref):
    x = (jnp.dot(h_ref[...], w1_ref[...], preferred_element_type=jnp.float32)
         + jnp.dot(uc_ref[...], w2_ref[...], preferred_element_type=jnp.float32)
         + b_ref[...])
    mu = jnp.mean(x, axis=1, keepdims=True)
    var = jnp.mean((x - mu) ** 2, axis=1, keepdims=True)
    y = (x - mu) / jnp.sqrt(var + 1e-5) * g_ref[...] + bt_ref[...]
    y = jnp.maximum(y, 0.0)
    o0_ref[...] = y[:, :HF]
    o1_ref[...] = y[:, HF:]


def _fin_body(h_ref, a0_ref, a1_ref, a2_ref, a3_ref, dg_ref,
              ws_ref, wn_ref, bs_ref, wg1_ref, wg2_ref, bg_ref, o_ref):
    deg = dg_ref[...][:, 0:1]
    inv = 1.0 / jnp.where(deg > 0.0, deg, 1.0)
    h = h_ref[...]
    hn = jnp.concatenate([a0_ref[...], a1_ref[...]], axis=1) * inv
    conv = jnp.concatenate([a2_ref[...], a3_ref[...]], axis=1) * inv
    std = (jnp.dot(h, ws_ref[...], preferred_element_type=jnp.float32)
           + jnp.dot(hn, wn_ref[...], preferred_element_type=jnp.float32)
           + bs_ref[...])
    gate = jax.nn.sigmoid(
        jnp.dot(std, wg1_ref[...], preferred_element_type=jnp.float32)
        + jnp.dot(conv, wg2_ref[...], preferred_element_type=jnp.float32)
        + bg_ref[...])
    o_ref[...] = gate * std + (1.0 - gate) * conv


SUP = 8


def _sc_scatter(src2, dst2, t0, t1, t2, t3, zrows, zdeg, ones8,
                *, n_pad, k_chunks):
    rpt = n_pad // NS
    n_sup = k_chunks // SUP
    mesh = plsc.VectorSubcoreMesh(core_axis_name="c", subcore_axis_name="s")
    out_sds = jax.ShapeDtypeStruct((n_pad, HF), jnp.float32)

    @functools.partial(
        pl.kernel,
        mesh=mesh,
        out_type=[out_sds, out_sds, out_sds, out_sds,
                  jax.ShapeDtypeStruct((n_pad, 8), jnp.float32)],
        scratch_types=[
            pltpu.VMEM((SUP, CHUNK), jnp.int32),
            pltpu.VMEM((SUP, CHUNK), jnp.int32),
            pltpu.VMEM((CHUNK, HF), jnp.float32),
            pltpu.VMEM((CHUNK, HF), jnp.float32),
            pltpu.VMEM((CHUNK, 8), jnp.float32),
            pltpu.VMEM((rpt, HF), jnp.float32),
            pltpu.VMEM((rpt, 8), jnp.float32),
            pltpu.VMEM_SHARED((n_pad, HF), jnp.float32),
            pltpu.VMEM_SHARED((n_pad, 8), jnp.float32),
            pltpu.SemaphoreType.DMA,
            pltpu.SemaphoreType.DMA,
            pltpu.SemaphoreType.DMA,
            pltpu.SemaphoreType.DMA,
            pltpu.SemaphoreType.DMA,
            pltpu.SemaphoreType.DMA,
        ],
        compiler_params=pltpu.CompilerParams(use_tc_tiling_on_sc=False),
    )
    def k(src_hbm, dst_hbm, t0_hbm, t1_hbm, t2_hbm, t3_hbm,
          zrows_hbm, zdeg_hbm, ones_hbm,
          out0, out1, out2, out3, out_deg,
          sbuf, dbuf, rows0, rows1, ones_v, zbuf_v, degbuf_v, acc_sh, deg_sh,
          gs0, gs1, ss0, ss1, ds0, ds1):
        cid = lax.axis_index("c")
        sid = lax.axis_index("s")
        row0 = sid * rpt
        rows = (rows0, rows1)
        gsem = (gs0, gs1)
        ssem = (ss0, ss1)
        dsem = (ds0, ds1)
        pltpu.sync_copy(ones_hbm, ones_v)
        pltpu.sync_copy(zdeg_hbm, degbuf_v)
        pltpu.sync_copy(degbuf_v, deg_sh.at[pl.ds(row0, rpt)])

        for p, (ta, tb, out) in enumerate(((t0_hbm, t2_hbm, (out0, out2)),
                                           (t1_hbm, t3_hbm, (out1, out3)))):
            pltpu.sync_copy(zrows_hbm, zbuf_v)
            pltpu.sync_copy(zbuf_v, acc_sh.at[pl.ds(row0, rpt)])
            plsc.subcore_barrier()

            def gather_fire(j, b, ta=ta, tb=tb):
                @pl.when(cid == 0)
                def _():
                    pltpu.async_copy(ta.at[sbuf.at[j]], rows[b], gsem[b])

                @pl.when(cid == 1)
                def _():
                    pltpu.async_copy(tb.at[sbuf.at[j]], rows[b], gsem[b])

            def gather_wait(j, b, ta=ta):
                pltpu.make_async_copy(ta.at[sbuf.at[j]], rows[b],
                                      gsem[b]).wait()

            def deg_fire(j, b):
                @pl.when(cid == 0)
                def _():
                    pltpu.async_copy(ones_v, deg_sh.at[dbuf.at[j]], dsem[b],
                                     add=True)

            def deg_wait(j, b):
                @pl.when(cid == 0)
                def _():
                    pltpu.make_async_copy(ones_v, deg_sh.at[dbuf.at[j]],
                                          dsem[b]).wait()

            def body(sb, carry, p=p):
                r0 = sid * k_chunks + sb * SUP
                pltpu.sync_copy(src_hbm.at[pl.ds(r0, SUP)], sbuf)
                pltpu.sync_copy(dst_hbm.at[pl.ds(r0, SUP)], dbuf)
                scat = [None, None]
                gather_fire(0, 0)
                for j in range(SUP):
                    b = j % 2
                    o = 1 - b
                    gather_wait(j, b)
                    scat[b] = pltpu.async_copy(
                        rows[b], acc_sh.at[dbuf.at[j]], ssem[b], add=True)
                    if p == 0:
                        deg_fire(j, b)
                    if j + 1 < SUP:
                        if scat[o] is not None:
                            scat[o].wait()
                            if p == 0:
                                deg_wait(j - 1, o)
                        gather_fire(j + 1, o)
                scat[0].wait()
                scat[1].wait()
                if p == 0:
                    deg_wait(SUP - 2, 0)
                    deg_wait(SUP - 1, 1)
                return carry

            lax.fori_loop(0, n_sup, body, 0)
            plsc.subcore_barrier()

            pltpu.sync_copy(acc_sh.at[pl.ds(row0, rpt)], zbuf_v)

            @pl.when(cid == 0)
            def _(out=out):
                pltpu.sync_copy(zbuf_v, out[0].at[pl.ds(row0, rpt)])

            @pl.when(cid == 1)
            def _(out=out):
                pltpu.sync_copy(zbuf_v, out[1].at[pl.ds(row0, rpt)])

            if p == 0:
                @pl.when(cid == 0)
                def _():
                    pltpu.sync_copy(deg_sh.at[pl.ds(row0, rpt)], degbuf_v)
                    pltpu.sync_copy(degbuf_v, out_deg.at[pl.ds(row0, rpt)])
                plsc.subcore_barrier()

    return k(src2, dst2, t0, t1, t2, t3, zrows, zdeg, ones8)


def kernel(h, edge_index, user_context_features, W_self, W_neigh, b_sage,
           W_msg, b_msg, ln_gamma, ln_beta, W_gate, b_gate):
    n, f = h.shape
    e = edge_index.shape[1]
    assert f == 128

    k_chunks = -(-(-(-e // (NS * CHUNK))) // SUP) * SUP
    e_pad = NS * CHUNK * k_chunks
    src = edge_index[0].astype(jnp.int32)
    dst = edge_index[1].astype(jnp.int32)
    if e_pad > e:
        pad = e_pad - e
        src = jnp.concatenate([src, jnp.zeros((pad,), jnp.int32)])
        dst = jnp.concatenate(
            [dst, n + (jnp.arange(pad, dtype=jnp.int32) % 128)])
    src2 = src.reshape(e_pad // CHUNK, CHUNK)
    dst2 = dst.reshape(e_pad // CHUNK, CHUNK)
    n_pad = -(-(n + 128) // 128) * 128
    rpt = n_pad // NS

    blk = 1000
    grid = n // blk
    full = lambda r, c: pl.BlockSpec((r, c), lambda i: (0, 0))
    rowsb = lambda c: pl.BlockSpec((blk, c), lambda i: (i, 0))
    m0, m1 = pl.pallas_call(
        _msg_body,
        grid=(grid,),
        in_specs=[rowsb(128), rowsb(128), full(128, 128), full(128, 128),
                  full(1, 128), full(1, 128), full(1, 128)],
        out_specs=[rowsb(HF), rowsb(HF)],
        out_shape=[jax.ShapeDtypeStruct((n, HF), jnp.float32)] * 2,
    )(h, user_context_features, W_msg[:128], W_msg[128:],
      b_msg.reshape(1, 128), ln_gamma.reshape(1, 128), ln_beta.reshape(1, 128))

    zrows = jnp.zeros((rpt, HF), jnp.float32)
    zdeg = jnp.zeros((rpt, 8), jnp.float32)
    ones8 = jnp.ones((CHUNK, 8), jnp.float32)
    a0, a1, a2, a3, deg2 = _sc_scatter(
        src2, dst2, h[:, :HF], h[:, HF:], m0, m1, zrows, zdeg, ones8,
        n_pad=n_pad, k_chunks=k_chunks)

    out = pl.pallas_call(
        _fin_body,
        grid=(grid,),
        in_specs=[rowsb(128), rowsb(HF), rowsb(HF), rowsb(HF), rowsb(HF),
                  rowsb(8),
                  full(128, 128), full(128, 128), full(1, 128),
                  full(128, 128), full(128, 128), full(1, 128)],
        out_specs=rowsb(128),
        out_shape=jax.ShapeDtypeStruct((n, 128), jnp.float32),
    )(h, a0, a1, a2, a3, deg2,
      W_self, W_neigh, b_sage.reshape(1, 128),
      W_gate[:128], W_gate[128:], b_gate.reshape(1, 128))
    return out
